# baseline (speedup 1.0000x reference)
import functools

import jax
import jax.numpy as jnp
from jax import lax
from jax.experimental import pallas as pl
from jax.experimental.pallas import tpu as pltpu
from jax.experimental.pallas import tpu_sc as plsc

_MAX_SIZE = 32
_L = 16


def _sc_gather_body(emb_hbm, rows_hbm, idx2, rows_v, gsem, *, W):
    half = W // 2

    @pl.when((lax.axis_index("c") == 0) & (lax.axis_index("s") == 0))
    def _():
        for r in range(2):
            for t in range(half // _L):
                base = r * half + t * _L
                iv = lax.iota(jnp.int32, _L) + (base - W // 2)
                idx2[r, pl.ds(t * _L, _L)] = (
                    jnp.clip(iv, -_MAX_SIZE, _MAX_SIZE) + _MAX_SIZE
                )

        pltpu.async_copy(
            emb_hbm.at[idx2.at[0]], rows_v.at[pl.ds(0, half)], gsem
        ).wait()
        pltpu.async_copy(
            emb_hbm.at[idx2.at[1]], rows_v.at[pl.ds(half, half)], gsem
        ).wait()
        pltpu.sync_copy(rows_v, rows_hbm)


def _clipped_onehot(n_rows, n_idx, base, center):
    row = lax.broadcasted_iota(jnp.int32, (n_rows, n_idx), 0)
    col = lax.broadcasted_iota(jnp.int32, (n_rows, n_idx), 1)
    idx = jnp.clip(base + row - center, -_MAX_SIZE, _MAX_SIZE) + _MAX_SIZE
    return (idx == col).astype(jnp.float32)


def _pos_a_kernel(emb_ref, out_ref, *, th, h, w, c):
    i = pl.program_id(0)
    n = emb_ref.shape[0]
    emb = emb_ref[...]
    oh_w = _clipped_onehot(w, n, 0, w // 2)
    rows_w = jnp.dot(oh_w, emb, preferred_element_type=jnp.float32)
    oh_h = _clipped_onehot(th, n, i * th, h // 2)
    rows_h = jnp.dot(oh_h, emb, preferred_element_type=jnp.float32)
    for j in range(th):
        out_ref[0, pl.ds(j * w, w), :] = rows_h[j : j + 1, :] + rows_w


def _pos_b_kernel(rows_w_ref, rows_h_ref, alias_ref, out_ref, *, th, w, c):
    del alias_ref
    rows_w = rows_w_ref[...][:, :c]
    rows_h = rows_h_ref[...][:, :c]
    for j in range(th):
        out_ref[0, pl.ds(j * w, w), :] = rows_h[j : j + 1, :] + rows_w


def kernel(feature_map, embedding):
    B, C, H, W = feature_map.shape
    mesh = plsc.VectorSubcoreMesh(core_axis_name="c", subcore_axis_name="s")
    sc_gather = pl.kernel(
        functools.partial(_sc_gather_body, W=W),
        out_type=jax.ShapeDtypeStruct((W, 128), jnp.float32),
        mesh=mesh,
        scratch_types=[
            pltpu.VMEM((2, W // 2), jnp.int32),
            pltpu.VMEM((W, 128), jnp.float32),
            pltpu.SemaphoreType.DMA,
        ],
    )
    rows128 = sc_gather(jnp.pad(embedding, ((0, 0), (0, 128 - C))))

    TH = 16
    half_steps = H // TH // 2
    pos_a = pl.pallas_call(
        functools.partial(_pos_a_kernel, th=TH, h=H, w=W, c=C),
        grid=(half_steps,),
        in_specs=[
            pl.BlockSpec((embedding.shape[0], C), lambda i: (0, 0)),
        ],
        out_specs=pl.BlockSpec((1, TH * W, C), lambda i: (0, i, 0)),
        out_shape=jax.ShapeDtypeStruct((1, H * W, C), jnp.float32),
        compiler_params=pltpu.CompilerParams(
            dimension_semantics=("arbitrary",),
        ),
    )(embedding)

    pos = pl.pallas_call(
        functools.partial(_pos_b_kernel, th=TH, w=W, c=C),
        grid=(half_steps,),
        in_specs=[
            pl.BlockSpec((W, 128), lambda i: (0, 0)),
            pl.BlockSpec((TH, 128), lambda i: (i + half_steps, 0)),
            pl.BlockSpec(memory_space=pltpu.MemorySpace.HBM),
        ],
        out_specs=pl.BlockSpec((1, TH * W, C), lambda i: (0, i + half_steps, 0)),
        out_shape=jax.ShapeDtypeStruct((1, H * W, C), jnp.float32),
        input_output_aliases={2: 0},
        compiler_params=pltpu.CompilerParams(
            dimension_semantics=("arbitrary",),
        ),
    )(rows128, rows128, pos_a)
    return jnp.broadcast_to(pos, (B, H * W, C))


# speedup vs baseline: 1.0931x; 1.0931x over previous
import functools

import jax
import jax.numpy as jnp
from jax import lax
from jax.experimental import pallas as pl
from jax.experimental.pallas import tpu as pltpu
from jax.experimental.pallas import tpu_sc as plsc

_MAX_SIZE = 32
_L = 16


def _sc_gather_body(emb_hbm, rows_hbm, idx2, rows_v, gsem, *, W):
    half = W // 2

    @pl.when((lax.axis_index("c") == 0) & (lax.axis_index("s") == 0))
    def _():
        for r in range(2):
            for t in range(half // _L):
                base = r * half + t * _L
                iv = lax.iota(jnp.int32, _L) + (base - W // 2)
                idx2[r, pl.ds(t * _L, _L)] = (
                    jnp.clip(iv, -_MAX_SIZE, _MAX_SIZE) + _MAX_SIZE
                )

        c1 = pltpu.async_copy(
            emb_hbm.at[idx2.at[0]], rows_v.at[pl.ds(0, half)], gsem
        )
        c2 = pltpu.async_copy(
            emb_hbm.at[idx2.at[1]], rows_v.at[pl.ds(half, half)], gsem
        )
        c1.wait()
        c2.wait()
        pltpu.sync_copy(rows_v, rows_hbm)


def _pos_kernel(rows_w_ref, rows_h_ref, out_ref, *, th, w, c):
    rows_w = rows_w_ref[...][:, :c]
    rows_h = rows_h_ref[...][:, :c]
    for j in range(th):
        out_ref[0, pl.ds(j * w, w), :] = rows_h[j : j + 1, :] + rows_w


def kernel(feature_map, embedding):
    B, C, H, W = feature_map.shape
    mesh = plsc.VectorSubcoreMesh(
        core_axis_name="c", subcore_axis_name="s", num_cores=1
    )
    sc_gather = pl.kernel(
        functools.partial(_sc_gather_body, W=W),
        out_type=jax.ShapeDtypeStruct((W, 128), jnp.float32),
        mesh=mesh,
        scratch_types=[
            pltpu.VMEM((2, W // 2), jnp.int32),
            pltpu.VMEM((W, 128), jnp.float32),
            pltpu.SemaphoreType.DMA,
        ],
    )
    rows128 = sc_gather(jnp.pad(embedding, ((0, 0), (0, 128 - C))))

    TH = 32
    pos = pl.pallas_call(
        functools.partial(_pos_kernel, th=TH, w=W, c=C),
        grid=(H // TH,),
        in_specs=[
            pl.BlockSpec((W, 128), lambda i: (0, 0)),
            pl.BlockSpec((TH, 128), lambda i: (i, 0)),
        ],
        out_specs=pl.BlockSpec((1, TH * W, C), lambda i: (0, i, 0)),
        out_shape=jax.ShapeDtypeStruct((1, H * W, C), jnp.float32),
        compiler_params=pltpu.CompilerParams(
            dimension_semantics=("parallel",),
        ),
    )(rows128, rows128)
    return jnp.broadcast_to(pos, (B, H * W, C))
